# baseline (speedup 1.0000x reference)
import jax
import jax.numpy as jnp
from jax import lax
from jax.experimental import pallas as pl

B = 512
N = 2048
D = 64
H = 64
M = 50
HOR2 = 60

TBA = 1
TBB = 8


def _kernel_a(feat, cand, mk,
              pW1, pb1, pg, pbn, pW2T, pb2,
              mW1, mb1, mg, mbn, mW2T, mb2,
              tp_o, off_o, p1_o):
    f = feat[...].reshape(TBA, D)
    fb = jnp.broadcast_to(f[:, None, :], (TBA, N, D)).reshape(TBA * N, D)
    x = jnp.concatenate([fb, cand[...]], axis=-1)

    def red64(a):
        acc = a[:, 0:8]
        for k in range(1, 8):
            acc = acc + a[:, 8 * k:8 * k + 8]
        t = acc[:, 0:4] + acc[:, 4:8]
        t = t[:, 0:2] + t[:, 2:4]
        return t[:, 0:1] + t[:, 1:2]

    def mlp(W1, b1, g, bn, W2, b2):
        h = x @ W1[...] + b1[...]
        mu = red64(h) * (1.0 / H)
        var = red64((h - mu) ** 2) * (1.0 / H)
        h = (h - mu) / jnp.sqrt(var + 1e-5) * g[...] + bn[...]
        h = jnp.maximum(h, 0.0)
        return h @ W2[...] + b2[...]

    logits = mlp(pW1, pb1, pg, pbn, pW2T, pb2)
    m = mk[...]
    l = logits * m
    mx = jnp.max(l, axis=-1, keepdims=True)
    e = jnp.exp(l - mx)
    s = jnp.sum(e, axis=-1, keepdims=True)
    p = e / s * m
    tp = p / (jnp.sum(p, axis=-1, keepdims=True) + 1e-13)
    tp_o[...] = tp
    p1_o[...] = tp[:, 1:2]
    off_o[...] = mlp(mW1, mb1, mg, mbn, mW2T, mb2)


def _kernel_b(p1, cx, cy, ox, oy, feat, gt,
              mot_W1, mot_b1, mot_g, mot_bn, mot_W2, mot_b2,
              sc_W1, sc_b1, sc_g, sc_bn, sc_W2, sc_b2,
              trajs_o, score_o, tg_o):
    p = p1[...]
    lx = cx[...] + ox[...]
    ly = cy[...] + oy[...]
    iotan = lax.broadcasted_iota(jnp.int32, (TBB, N), 1)
    ci = lax.broadcasted_iota(jnp.int32, (TBB, M, 2), 1)

    def step(i, carry):
        p, acc = carry
        mx = jnp.max(p, axis=-1, keepdims=True)
        idx = jnp.min(jnp.where(p == mx, iotan, N), axis=-1, keepdims=True)
        oh = iotan == idx
        gx = jnp.sum(jnp.where(oh, lx, 0.0), axis=-1, keepdims=True)
        gy = jnp.sum(jnp.where(oh, ly, 0.0), axis=-1, keepdims=True)
        g3 = jnp.concatenate([gx, gy], axis=-1)[:, None, :]
        acc = jnp.where(ci == i, g3, acc)
        p = jnp.where(oh, -jnp.inf, p)
        return p, acc

    _, loc3 = lax.fori_loop(
        0, M, step, (p, jnp.zeros((TBB, M, 2), jnp.float32)))

    f = feat[...]
    f3 = jnp.broadcast_to(f[:, None, :], (TBB, M, D))

    def mlp2(x, W1, b1, g, bn, W2, b2):
        h = x @ W1[...] + b1[...]
        mu = jnp.mean(h, axis=-1, keepdims=True)
        var = jnp.mean((h - mu) ** 2, axis=-1, keepdims=True)
        h = (h - mu) / jnp.sqrt(var + 1e-5) * g[...] + bn[...]
        h = jnp.maximum(h, 0.0)
        return h @ W2[...] + b2[...]

    xm = jnp.concatenate([f3, loc3], axis=-1).reshape(TBB * M, D + 2)
    trj = mlp2(xm, mot_W1, mot_b1, mot_g, mot_bn, mot_W2, mot_b2)
    trj3 = trj.reshape(TBB, M, HOR2)
    trajs_o[...] = trj3

    xs = jnp.concatenate([f3, trj3], axis=-1).reshape(TBB * M, D + HOR2)
    sl = mlp2(xs, sc_W1, sc_b1, sc_g, sc_bn, sc_W2, sc_b2)
    s3 = sl.reshape(TBB, M, 1)
    smx = jnp.max(s3, axis=1, keepdims=True)
    es = jnp.exp(s3 - smx)
    score_o[...] = es / jnp.sum(es, axis=1, keepdims=True)

    xg = jnp.concatenate([f, gt[...]], axis=-1)
    tg_o[...] = mlp2(xg, mot_W1, mot_b1, mot_g, mot_bn, mot_W2, mot_b2)


def kernel(target_feat, target_candidate, candidate_mask, target_gt,
           prob_W1, prob_b1, prob_g, prob_bn, prob_W2, prob_b2,
           mean_W1, mean_b1, mean_g, mean_bn, mean_W2, mean_b2,
           mot_W1, mot_b1, mot_g, mot_bn, mot_W2, mot_b2,
           sc_W1, sc_b1, sc_g, sc_bn, sc_W2, sc_b2):
    feat2 = target_feat.reshape(B, D)
    cand2 = target_candidate.reshape(B * N, 2)
    maskc = candidate_mask.astype(jnp.float32).reshape(B * N, 1)

    r1 = lambda a: a.reshape(1, -1)
    wspec = lambda shp: pl.BlockSpec(shp, lambda i: (0, 0))

    tp2, off2, p1c = pl.pallas_call(
        _kernel_a,
        grid=(B // TBA,),
        in_specs=[
            pl.BlockSpec((1, TBA, D), lambda i: (i, 0, 0)),
            pl.BlockSpec((TBA * N, 2), lambda i: (i, 0)),
            pl.BlockSpec((TBA * N, 1), lambda i: (i, 0)),
            wspec((D + 2, H)), wspec((1, H)), wspec((1, H)), wspec((1, H)),
            wspec((H, 2)), wspec((1, 2)),
            wspec((D + 2, H)), wspec((1, H)), wspec((1, H)), wspec((1, H)),
            wspec((H, 2)), wspec((1, 2)),
        ],
        out_specs=[
            pl.BlockSpec((TBA * N, 2), lambda i: (i, 0)),
            pl.BlockSpec((TBA * N, 2), lambda i: (i, 0)),
            pl.BlockSpec((TBA * N, 1), lambda i: (i, 0)),
        ],
        out_shape=[
            jax.ShapeDtypeStruct((B * N, 2), jnp.float32),
            jax.ShapeDtypeStruct((B * N, 2), jnp.float32),
            jax.ShapeDtypeStruct((B * N, 1), jnp.float32),
        ],
    )(feat2.reshape(B // TBA, TBA, D), cand2, maskc,
      prob_W1, r1(prob_b1), r1(prob_g), r1(prob_bn), prob_W2, r1(prob_b2),
      mean_W1, r1(mean_b1), r1(mean_g), r1(mean_bn), mean_W2, r1(mean_b2))

    target_prob = tp2.reshape(B, N, 2)
    offset = off2.reshape(B, N, 2)
    p1 = p1c.reshape(B, N)
    cx = target_candidate[..., 0]
    cy = target_candidate[..., 1]
    ox = offset[..., 0]
    oy = offset[..., 1]
    gt2 = target_gt.reshape(B, 2)

    bspec = lambda: pl.BlockSpec((TBB, N), lambda i: (i, 0))
    trajs, score3, tg = pl.pallas_call(
        _kernel_b,
        grid=(B // TBB,),
        in_specs=[
            bspec(), bspec(), bspec(), bspec(), bspec(),
            pl.BlockSpec((TBB, D), lambda i: (i, 0)),
            pl.BlockSpec((TBB, 2), lambda i: (i, 0)),
            wspec((D + 2, H)), wspec((1, H)), wspec((1, H)), wspec((1, H)),
            wspec((H, HOR2)), wspec((1, HOR2)),
            wspec((D + HOR2, H)), wspec((1, H)), wspec((1, H)), wspec((1, H)),
            wspec((H, 1)), wspec((1, 1)),
        ],
        out_specs=[
            pl.BlockSpec((TBB, M, HOR2), lambda i: (i, 0, 0)),
            pl.BlockSpec((TBB, M, 1), lambda i: (i, 0, 0)),
            pl.BlockSpec((TBB, HOR2), lambda i: (i, 0)),
        ],
        out_shape=[
            jax.ShapeDtypeStruct((B, M, HOR2), jnp.float32),
            jax.ShapeDtypeStruct((B, M, 1), jnp.float32),
            jax.ShapeDtypeStruct((B, HOR2), jnp.float32),
        ],
    )(p1, cx, cy, ox, oy, feat2, gt2,
      mot_W1, r1(mot_b1), r1(mot_g), r1(mot_bn), mot_W2, r1(mot_b2),
      sc_W1, r1(sc_b1), r1(sc_g), r1(sc_bn), sc_W2, r1(sc_b2))

    return (target_prob, offset, tg.reshape(B, 1, HOR2),
            trajs, score3.reshape(B, M))


# speedup vs baseline: 1.0311x; 1.0311x over previous
import jax
import jax.numpy as jnp
from jax import lax
from jax.experimental import pallas as pl

B = 512
N = 2048
D = 64
H = 64
M = 50
HOR2 = 60

TBA = 1
TBB = 8


def _kernel_a(feat, cand, mk,
              pW1, pb1, pg, pbn, pW2T, pb2,
              mW1, mb1, mg, mbn, mW2T, mb2,
              tp_o, off_o, p1_o, cx_o, cy_o, ox_o, oy_o):
    f = feat[...].reshape(TBA, D)
    fb = jnp.broadcast_to(f[:, None, :], (TBA, N, D)).reshape(TBA * N, D)
    x = jnp.concatenate([fb, cand[...]], axis=-1)

    def red64(a):
        acc = a[:, 0:8]
        for k in range(1, 8):
            acc = acc + a[:, 8 * k:8 * k + 8]
        t = acc[:, 0:4] + acc[:, 4:8]
        t = t[:, 0:2] + t[:, 2:4]
        return t[:, 0:1] + t[:, 1:2]

    def mlp(W1, b1, g, bn, W2, b2):
        h = x @ W1[...] + b1[...]
        mu = red64(h) * (1.0 / H)
        var = red64((h - mu) ** 2) * (1.0 / H)
        h = (h - mu) / jnp.sqrt(var + 1e-5) * g[...] + bn[...]
        h = jnp.maximum(h, 0.0)
        return h @ W2[...] + b2[...]

    logits = mlp(pW1, pb1, pg, pbn, pW2T, pb2)
    m = mk[...]
    l = logits * m
    mx = jnp.max(l, axis=-1, keepdims=True)
    e = jnp.exp(l - mx)
    s = jnp.sum(e, axis=-1, keepdims=True)
    p = e / s * m
    tp = p / (jnp.sum(p, axis=-1, keepdims=True) + 1e-13)
    tp_o[...] = tp
    p1_o[...] = tp[:, 1:2]
    off = mlp(mW1, mb1, mg, mbn, mW2T, mb2)
    off_o[...] = off
    c = cand[...]
    cx_o[...] = c[:, 0:1]
    cy_o[...] = c[:, 1:2]
    ox_o[...] = off[:, 0:1]
    oy_o[...] = off[:, 1:2]


def _kernel_b(p1, cx, cy, ox, oy, feat, gt,
              mot_W1, mot_b1, mot_g, mot_bn, mot_W2, mot_b2,
              sc_W1, sc_b1, sc_g, sc_bn, sc_W2, sc_b2,
              trajs_o, score_o, tg_o):
    p = p1[...]
    lx = cx[...] + ox[...]
    ly = cy[...] + oy[...]
    iotan = lax.broadcasted_iota(jnp.int32, (TBB, N), 1)
    ci = lax.broadcasted_iota(jnp.int32, (TBB, M, 2), 1)

    def step(i, carry):
        p, acc = carry
        mx = jnp.max(p, axis=-1, keepdims=True)
        idx = jnp.min(jnp.where(p == mx, iotan, N), axis=-1, keepdims=True)
        oh = iotan == idx
        gx = jnp.sum(jnp.where(oh, lx, 0.0), axis=-1, keepdims=True)
        gy = jnp.sum(jnp.where(oh, ly, 0.0), axis=-1, keepdims=True)
        g3 = jnp.concatenate([gx, gy], axis=-1)[:, None, :]
        acc = jnp.where(ci == i, g3, acc)
        p = jnp.where(oh, -jnp.inf, p)
        return p, acc

    _, loc3 = lax.fori_loop(
        0, M, step, (p, jnp.zeros((TBB, M, 2), jnp.float32)))

    f = feat[...]
    f3 = jnp.broadcast_to(f[:, None, :], (TBB, M, D))

    def mlp2(x, W1, b1, g, bn, W2, b2):
        h = x @ W1[...] + b1[...]
        mu = jnp.mean(h, axis=-1, keepdims=True)
        var = jnp.mean((h - mu) ** 2, axis=-1, keepdims=True)
        h = (h - mu) / jnp.sqrt(var + 1e-5) * g[...] + bn[...]
        h = jnp.maximum(h, 0.0)
        return h @ W2[...] + b2[...]

    xm = jnp.concatenate([f3, loc3], axis=-1).reshape(TBB * M, D + 2)
    trj = mlp2(xm, mot_W1, mot_b1, mot_g, mot_bn, mot_W2, mot_b2)
    trj3 = trj.reshape(TBB, M, HOR2)
    trajs_o[...] = trj3

    xs = jnp.concatenate([f3, trj3], axis=-1).reshape(TBB * M, D + HOR2)
    sl = mlp2(xs, sc_W1, sc_b1, sc_g, sc_bn, sc_W2, sc_b2)
    s3 = sl.reshape(TBB, M, 1)
    smx = jnp.max(s3, axis=1, keepdims=True)
    es = jnp.exp(s3 - smx)
    score_o[...] = es / jnp.sum(es, axis=1, keepdims=True)

    xg = jnp.concatenate([f, gt[...]], axis=-1)
    tg_o[...] = mlp2(xg, mot_W1, mot_b1, mot_g, mot_bn, mot_W2, mot_b2)


def kernel(target_feat, target_candidate, candidate_mask, target_gt,
           prob_W1, prob_b1, prob_g, prob_bn, prob_W2, prob_b2,
           mean_W1, mean_b1, mean_g, mean_bn, mean_W2, mean_b2,
           mot_W1, mot_b1, mot_g, mot_bn, mot_W2, mot_b2,
           sc_W1, sc_b1, sc_g, sc_bn, sc_W2, sc_b2):
    feat2 = target_feat.reshape(B, D)
    cand2 = target_candidate.reshape(B * N, 2)
    maskc = candidate_mask.astype(jnp.float32).reshape(B * N, 1)

    r1 = lambda a: a.reshape(1, -1)
    wspec = lambda shp: pl.BlockSpec(shp, lambda i: (0, 0))

    tp2, off2, p1c, cxc, cyc, oxc, oyc = pl.pallas_call(
        _kernel_a,
        grid=(B // TBA,),
        in_specs=[
            pl.BlockSpec((1, TBA, D), lambda i: (i, 0, 0)),
            pl.BlockSpec((TBA * N, 2), lambda i: (i, 0)),
            pl.BlockSpec((TBA * N, 1), lambda i: (i, 0)),
            wspec((D + 2, H)), wspec((1, H)), wspec((1, H)), wspec((1, H)),
            wspec((H, 2)), wspec((1, 2)),
            wspec((D + 2, H)), wspec((1, H)), wspec((1, H)), wspec((1, H)),
            wspec((H, 2)), wspec((1, 2)),
        ],
        out_specs=[
            pl.BlockSpec((TBA * N, 2), lambda i: (i, 0)),
            pl.BlockSpec((TBA * N, 2), lambda i: (i, 0)),
        ] + [pl.BlockSpec((TBA * N, 1), lambda i: (i, 0))] * 5,
        out_shape=[
            jax.ShapeDtypeStruct((B * N, 2), jnp.float32),
            jax.ShapeDtypeStruct((B * N, 2), jnp.float32),
        ] + [jax.ShapeDtypeStruct((B * N, 1), jnp.float32)] * 5,
    )(feat2.reshape(B // TBA, TBA, D), cand2, maskc,
      prob_W1, r1(prob_b1), r1(prob_g), r1(prob_bn), prob_W2, r1(prob_b2),
      mean_W1, r1(mean_b1), r1(mean_g), r1(mean_bn), mean_W2, r1(mean_b2))

    target_prob = tp2.reshape(B, N, 2)
    offset = off2.reshape(B, N, 2)
    p1 = p1c.reshape(B, N)
    cx = cxc.reshape(B, N)
    cy = cyc.reshape(B, N)
    ox = oxc.reshape(B, N)
    oy = oyc.reshape(B, N)
    gt2 = target_gt.reshape(B, 2)

    bspec = lambda: pl.BlockSpec((TBB, N), lambda i: (i, 0))
    trajs, score3, tg = pl.pallas_call(
        _kernel_b,
        grid=(B // TBB,),
        in_specs=[
            bspec(), bspec(), bspec(), bspec(), bspec(),
            pl.BlockSpec((TBB, D), lambda i: (i, 0)),
            pl.BlockSpec((TBB, 2), lambda i: (i, 0)),
            wspec((D + 2, H)), wspec((1, H)), wspec((1, H)), wspec((1, H)),
            wspec((H, HOR2)), wspec((1, HOR2)),
            wspec((D + HOR2, H)), wspec((1, H)), wspec((1, H)), wspec((1, H)),
            wspec((H, 1)), wspec((1, 1)),
        ],
        out_specs=[
            pl.BlockSpec((TBB, M, HOR2), lambda i: (i, 0, 0)),
            pl.BlockSpec((TBB, M, 1), lambda i: (i, 0, 0)),
            pl.BlockSpec((TBB, HOR2), lambda i: (i, 0)),
        ],
        out_shape=[
            jax.ShapeDtypeStruct((B, M, HOR2), jnp.float32),
            jax.ShapeDtypeStruct((B, M, 1), jnp.float32),
            jax.ShapeDtypeStruct((B, HOR2), jnp.float32),
        ],
    )(p1, cx, cy, ox, oy, feat2, gt2,
      mot_W1, r1(mot_b1), r1(mot_g), r1(mot_bn), mot_W2, r1(mot_b2),
      sc_W1, r1(sc_b1), r1(sc_g), r1(sc_bn), sc_W2, r1(sc_b2))

    return (target_prob, offset, tg.reshape(B, 1, HOR2),
            trajs, score3.reshape(B, M))


# speedup vs baseline: 1.1108x; 1.0773x over previous
import jax
import jax.numpy as jnp
from jax import lax
from jax.experimental import pallas as pl

B = 512
N = 2048
D = 64
H = 64
M = 50
HOR2 = 60

TBA = 1
TBB = 32


def _kernel_a(feat, cand, mk,
              pW1, pb1, pg, pbn, pW2T, pb2,
              mW1, mb1, mg, mbn, mW2T, mb2,
              tp_o, off_o, p1_o, cx_o, cy_o, ox_o, oy_o):
    f = feat[...].reshape(TBA, D)
    fb = jnp.broadcast_to(f[:, None, :], (TBA, N, D)).reshape(TBA * N, D)
    x = jnp.concatenate([fb, cand[...]], axis=-1)

    def red64(a):
        acc = a[:, 0:8]
        for k in range(1, 8):
            acc = acc + a[:, 8 * k:8 * k + 8]
        t = acc[:, 0:4] + acc[:, 4:8]
        t = t[:, 0:2] + t[:, 2:4]
        return t[:, 0:1] + t[:, 1:2]

    def mlp(W1, b1, g, bn, W2, b2):
        h = x @ W1[...] + b1[...]
        mu = red64(h) * (1.0 / H)
        var = red64((h - mu) ** 2) * (1.0 / H)
        h = (h - mu) / jnp.sqrt(var + 1e-5) * g[...] + bn[...]
        h = jnp.maximum(h, 0.0)
        return h @ W2[...] + b2[...]

    logits = mlp(pW1, pb1, pg, pbn, pW2T, pb2)
    m = mk[...]
    l = logits * m
    mx = jnp.max(l, axis=-1, keepdims=True)
    e = jnp.exp(l - mx)
    s = jnp.sum(e, axis=-1, keepdims=True)
    p = e / s * m
    tp = p / (jnp.sum(p, axis=-1, keepdims=True) + 1e-13)
    tp_o[...] = tp
    p1_o[...] = tp[:, 1:2]
    off = mlp(mW1, mb1, mg, mbn, mW2T, mb2)
    off_o[...] = off
    c = cand[...]
    cx_o[...] = c[:, 0:1]
    cy_o[...] = c[:, 1:2]
    ox_o[...] = off[:, 0:1]
    oy_o[...] = off[:, 1:2]


def _kernel_b(p1, cx, cy, ox, oy, feat, gt,
              mot_W1, mot_b1, mot_g, mot_bn, mot_W2, mot_b2,
              sc_W1, sc_b1, sc_g, sc_bn, sc_W2, sc_b2,
              trajs_o, score_o, tg_o):
    p = p1[...]
    lx = cx[...] + ox[...]
    ly = cy[...] + oy[...]
    iotan = lax.broadcasted_iota(jnp.int32, (TBB, N), 1)
    ci = lax.broadcasted_iota(jnp.int32, (TBB, M, 2), 1)

    def step(i, carry):
        p, acc = carry
        mx = jnp.max(p, axis=-1, keepdims=True)
        idx = jnp.min(jnp.where(p == mx, iotan, N), axis=-1, keepdims=True)
        oh = iotan == idx
        gx = jnp.sum(jnp.where(oh, lx, 0.0), axis=-1, keepdims=True)
        gy = jnp.sum(jnp.where(oh, ly, 0.0), axis=-1, keepdims=True)
        g3 = jnp.concatenate([gx, gy], axis=-1)[:, None, :]
        acc = jnp.where(ci == i, g3, acc)
        p = jnp.where(oh, -jnp.inf, p)
        return p, acc

    _, loc3 = lax.fori_loop(
        0, M, step, (p, jnp.zeros((TBB, M, 2), jnp.float32)))

    f = feat[...]
    f3 = jnp.broadcast_to(f[:, None, :], (TBB, M, D))

    def mlp2(x, W1, b1, g, bn, W2, b2):
        h = x @ W1[...] + b1[...]
        mu = jnp.mean(h, axis=-1, keepdims=True)
        var = jnp.mean((h - mu) ** 2, axis=-1, keepdims=True)
        h = (h - mu) / jnp.sqrt(var + 1e-5) * g[...] + bn[...]
        h = jnp.maximum(h, 0.0)
        return h @ W2[...] + b2[...]

    xm = jnp.concatenate([f3, loc3], axis=-1).reshape(TBB * M, D + 2)
    trj = mlp2(xm, mot_W1, mot_b1, mot_g, mot_bn, mot_W2, mot_b2)
    trj3 = trj.reshape(TBB, M, HOR2)
    trajs_o[...] = trj3

    xs = jnp.concatenate([f3, trj3], axis=-1).reshape(TBB * M, D + HOR2)
    sl = mlp2(xs, sc_W1, sc_b1, sc_g, sc_bn, sc_W2, sc_b2)
    s3 = sl.reshape(TBB, M, 1)
    smx = jnp.max(s3, axis=1, keepdims=True)
    es = jnp.exp(s3 - smx)
    score_o[...] = es / jnp.sum(es, axis=1, keepdims=True)

    xg = jnp.concatenate([f, gt[...]], axis=-1)
    tg_o[...] = mlp2(xg, mot_W1, mot_b1, mot_g, mot_bn, mot_W2, mot_b2)


def kernel(target_feat, target_candidate, candidate_mask, target_gt,
           prob_W1, prob_b1, prob_g, prob_bn, prob_W2, prob_b2,
           mean_W1, mean_b1, mean_g, mean_bn, mean_W2, mean_b2,
           mot_W1, mot_b1, mot_g, mot_bn, mot_W2, mot_b2,
           sc_W1, sc_b1, sc_g, sc_bn, sc_W2, sc_b2):
    feat2 = target_feat.reshape(B, D)
    cand2 = target_candidate.reshape(B * N, 2)
    maskc = candidate_mask.astype(jnp.float32).reshape(B * N, 1)

    r1 = lambda a: a.reshape(1, -1)
    wspec = lambda shp: pl.BlockSpec(shp, lambda i: (0, 0))

    tp2, off2, p1c, cxc, cyc, oxc, oyc = pl.pallas_call(
        _kernel_a,
        grid=(B // TBA,),
        in_specs=[
            pl.BlockSpec((1, TBA, D), lambda i: (i, 0, 0)),
            pl.BlockSpec((TBA * N, 2), lambda i: (i, 0)),
            pl.BlockSpec((TBA * N, 1), lambda i: (i, 0)),
            wspec((D + 2, H)), wspec((1, H)), wspec((1, H)), wspec((1, H)),
            wspec((H, 2)), wspec((1, 2)),
            wspec((D + 2, H)), wspec((1, H)), wspec((1, H)), wspec((1, H)),
            wspec((H, 2)), wspec((1, 2)),
        ],
        out_specs=[
            pl.BlockSpec((TBA * N, 2), lambda i: (i, 0)),
            pl.BlockSpec((TBA * N, 2), lambda i: (i, 0)),
        ] + [pl.BlockSpec((TBA * N, 1), lambda i: (i, 0))] * 5,
        out_shape=[
            jax.ShapeDtypeStruct((B * N, 2), jnp.float32),
            jax.ShapeDtypeStruct((B * N, 2), jnp.float32),
        ] + [jax.ShapeDtypeStruct((B * N, 1), jnp.float32)] * 5,
    )(feat2.reshape(B // TBA, TBA, D), cand2, maskc,
      prob_W1, r1(prob_b1), r1(prob_g), r1(prob_bn), prob_W2, r1(prob_b2),
      mean_W1, r1(mean_b1), r1(mean_g), r1(mean_bn), mean_W2, r1(mean_b2))

    target_prob = tp2.reshape(B, N, 2)
    offset = off2.reshape(B, N, 2)
    p1 = p1c.reshape(B, N)
    cx = cxc.reshape(B, N)
    cy = cyc.reshape(B, N)
    ox = oxc.reshape(B, N)
    oy = oyc.reshape(B, N)
    gt2 = target_gt.reshape(B, 2)

    bspec = lambda: pl.BlockSpec((TBB, N), lambda i: (i, 0))
    trajs, score3, tg = pl.pallas_call(
        _kernel_b,
        grid=(B // TBB,),
        in_specs=[
            bspec(), bspec(), bspec(), bspec(), bspec(),
            pl.BlockSpec((TBB, D), lambda i: (i, 0)),
            pl.BlockSpec((TBB, 2), lambda i: (i, 0)),
            wspec((D + 2, H)), wspec((1, H)), wspec((1, H)), wspec((1, H)),
            wspec((H, HOR2)), wspec((1, HOR2)),
            wspec((D + HOR2, H)), wspec((1, H)), wspec((1, H)), wspec((1, H)),
            wspec((H, 1)), wspec((1, 1)),
        ],
        out_specs=[
            pl.BlockSpec((TBB, M, HOR2), lambda i: (i, 0, 0)),
            pl.BlockSpec((TBB, M, 1), lambda i: (i, 0, 0)),
            pl.BlockSpec((TBB, HOR2), lambda i: (i, 0)),
        ],
        out_shape=[
            jax.ShapeDtypeStruct((B, M, HOR2), jnp.float32),
            jax.ShapeDtypeStruct((B, M, 1), jnp.float32),
            jax.ShapeDtypeStruct((B, HOR2), jnp.float32),
        ],
    )(p1, cx, cy, ox, oy, feat2, gt2,
      mot_W1, r1(mot_b1), r1(mot_g), r1(mot_bn), mot_W2, r1(mot_b2),
      sc_W1, r1(sc_b1), r1(sc_g), r1(sc_bn), sc_W2, r1(sc_b2))

    return (target_prob, offset, tg.reshape(B, 1, HOR2),
            trajs, score3.reshape(B, M))
